# baseline (speedup 1.0000x reference)
import functools

import jax
import jax.numpy as jnp
from jax import lax
from jax.experimental import pallas as pl
from jax.experimental.pallas import tpu as pltpu
from jax.experimental.pallas import tpu_sc as plsc

N = 5850
E = 93600
NP = 5888
RPT = 368
NW = 32
CH = 128
NCH = 23
EW = NCH * CH
EP = NW * EW
D1 = 16
D2 = 24

_sc_params = pltpu.CompilerParams(use_tc_tiling_on_sc=False)
_sc_params_nl = pltpu.CompilerParams(use_tc_tiling_on_sc=False,
                                     needs_layout_passes=False)


@functools.cache
def _mesh():
    return plsc.VectorSubcoreMesh(
        core_axis_name="c", subcore_axis_name="s", num_cores=2, num_subcores=16
    )


def _leaky(v):
    return jnp.where(v >= 0, v, 0.01 * v)


def _sc_layer1(h1, srcp, dstp):
    @functools.partial(
        pl.kernel,
        out_type=(
            jax.ShapeDtypeStruct((2, NP, D1), jnp.float32),
            jax.ShapeDtypeStruct((NP, D1), jnp.float32),
            jax.ShapeDtypeStruct((NP, D1), jnp.float32),
        ),
        mesh=_mesh(),
        compiler_params=_sc_params_nl,
        scratch_types=[
            pltpu.VMEM((NCH, CH), jnp.int32),
            pltpu.VMEM((NCH, CH), jnp.int32),
            pltpu.VMEM((CH, D1), jnp.float32),
            pltpu.VMEM((RPT, D1), jnp.float32),
            pltpu.VMEM((RPT, D1), jnp.float32),
            pltpu.VMEM((RPT, D1), jnp.float32),
            pltpu.VMEM((EW, D1), jnp.float32),
            pltpu.VMEM_SHARED((NP, D1), jnp.float32),
            pltpu.VMEM_SHARED((NP, D1), jnp.float32),
            pltpu.VMEM_SHARED((NP, D1), jnp.float32),
            pltpu.SemaphoreType.DMA,
            pltpu.SemaphoreType.DMA,
        ],
    )
    def k(h1_hbm, srcp_hbm, dstp_hbm, z_hbm, ones_hbm,
          s1_hbm, g1_hbm, dinv_hbm,
          idx_a, idx_b, onesb, degb, gb, hb, rows,
          acc_deg, g_sh, acc1, sem, sem2):
        c = lax.axis_index("c")
        s = lax.axis_index("s")
        wid = c * 16 + s
        row = pl.ds(s * RPT, RPT)
        pltpu.sync_copy(z_hbm, acc_deg.at[row, :])
        pltpu.sync_copy(z_hbm, acc1.at[row, :])
        pltpu.sync_copy(ones_hbm, onesb)
        pltpu.sync_copy(dstp_hbm.at[2 * s], idx_a)
        pltpu.sync_copy(dstp_hbm.at[2 * s + 1], idx_b)
        pltpu.sync_copy(h1_hbm.at[row, :], hb)
        plsc.subcore_barrier()
        adds = [
            pltpu.async_copy(onesb, acc_deg.at[idx_a.at[j]], sem, add=True)
            for j in range(NCH)
        ] + [
            pltpu.async_copy(onesb, acc_deg.at[idx_b.at[j]], sem, add=True)
            for j in range(NCH)
        ]
        for cp in adds:
            cp.wait()
        plsc.subcore_barrier()
        pltpu.sync_copy(acc_deg.at[row, :], degb)

        def newton(r, _):
            deg = degb[r, :] + 1.0
            i = plsc.bitcast(deg, jnp.int32)
            i = jnp.int32(0x5F3759DF) - lax.shift_right_logical(i, 1)
            y = plsc.bitcast(i, jnp.float32)
            half = -0.5 * deg
            for _it in range(4):
                y = y * (1.5 + half * y * y)
            degb[r, :] = y
            gb[r, :] = y * hb[r, :]
            return 0

        lax.fori_loop(0, RPT, newton, 0)
        pltpu.sync_copy(gb, g_sh.at[row, :])
        pltpu.sync_copy(gb, g1_hbm.at[row, :])
        pltpu.sync_copy(degb, dinv_hbm.at[row, :])
        pltpu.sync_copy(srcp_hbm.at[wid], idx_a)
        pltpu.sync_copy(dstp_hbm.at[wid], idx_b)
        plsc.subcore_barrier()
        cps = [
            pltpu.async_copy(g_sh.at[idx_a.at[j]], rows.at[pl.ds(j * CH, CH), :], sem)
            for j in range(NCH)
        ]
        for cp in cps:
            cp.wait()
        adds2 = [
            pltpu.async_copy(rows.at[pl.ds(j * CH, CH), :], acc1.at[idx_b.at[j]],
                             sem2, add=True)
            for j in range(NCH)
        ]
        for cp in adds2:
            cp.wait()
        plsc.subcore_barrier()
        pltpu.sync_copy(acc1.at[row, :], s1_hbm.at[c].at[row, :])

    return k(h1, srcp, dstp, jnp.zeros((RPT, D1), jnp.float32),
             jnp.ones((CH, D1), jnp.float32))


def _sc_scatter(g, srcp, dstp, d):
    @functools.partial(
        pl.kernel,
        out_type=jax.ShapeDtypeStruct((2, NP, d), jnp.float32),
        mesh=_mesh(),
        compiler_params=_sc_params,
        scratch_types=[
            pltpu.VMEM((NCH, CH), jnp.int32),
            pltpu.VMEM((NCH, CH), jnp.int32),
            pltpu.VMEM((EW, d), jnp.float32),
            pltpu.VMEM_SHARED((NP, d), jnp.float32),
            pltpu.VMEM_SHARED((NP, d), jnp.float32),
            pltpu.SemaphoreType.DMA,
            pltpu.SemaphoreType.DMA,
        ],
    )
    def k(g_hbm, srcp_hbm, dstp_hbm, z_hbm, out_hbm, idx_s, idx_d, rows, g_sh, acc,
          sem, sem2):
        c = lax.axis_index("c")
        s = lax.axis_index("s")
        wid = c * 16 + s
        row = pl.ds(s * RPT, RPT)
        pltpu.sync_copy(z_hbm, acc.at[row, :])
        pltpu.sync_copy(g_hbm.at[row, :], g_sh.at[row, :])
        pltpu.sync_copy(srcp_hbm.at[wid], idx_s)
        pltpu.sync_copy(dstp_hbm.at[wid], idx_d)
        plsc.subcore_barrier()
        cps = [
            pltpu.async_copy(g_sh.at[idx_s.at[j]], rows.at[pl.ds(j * CH, CH), :], sem)
            for j in range(NCH)
        ]
        for cp in cps:
            cp.wait()
        adds = [
            pltpu.async_copy(rows.at[pl.ds(j * CH, CH), :], acc.at[idx_d.at[j]],
                             sem2, add=True)
            for j in range(NCH)
        ]
        for cp in adds:
            cp.wait()
        plsc.subcore_barrier()
        pltpu.sync_copy(acc.at[row, :], out_hbm.at[c].at[row, :])

    return k(g, srcp, dstp, jnp.zeros((RPT, d), jnp.float32))


def _tc_h1(posp, W1):
    def body(pos_ref, w_ref, h1_ref):
        w = jnp.pad(w_ref[...], ((0, 0), (0, D1 - 5)))
        h1_ref[...] = jnp.dot(pos_ref[...], w, preferred_element_type=jnp.float32)

    return pl.pallas_call(
        body,
        out_shape=jax.ShapeDtypeStruct((NP, D1), jnp.float32),
    )(posp, W1)


def _tc_mid(s1p, g1, dinv16, b1, W2):
    def body(s_ref, g1_ref, dinv_ref, b1_ref, w2_ref, g2_ref):
        dinv = dinv_ref[:, 0:1]
        b1p = jnp.pad(b1_ref[...], (0, D1 - 5))
        w2p = jnp.pad(w2_ref[...], ((0, D1 - 5), (0, D2 - 20)))
        x1 = _leaky(dinv * (s_ref[0] + s_ref[1] + g1_ref[...]) + b1p)
        h2 = jnp.dot(x1, w2p, preferred_element_type=jnp.float32)
        g2_ref[...] = dinv * h2

    return pl.pallas_call(
        body,
        out_shape=jax.ShapeDtypeStruct((NP, D2), jnp.float32),
    )(s1p, g1, dinv16, b1, W2)


def _tc_post(s2p, g2, dinv16, b2, fc1_W, fc1_b):
    def body(s_ref, g2_ref, dinv_ref, b2_ref, w_ref, b_ref, x3_ref):
        b2p = jnp.pad(b2_ref[...], (0, D2 - 20))
        wp = jnp.pad(w_ref[...], ((0, D2 - 20), (0, 0)))
        x2 = _leaky(dinv_ref[:, 0:1] * (s_ref[0] + s_ref[1] + g2_ref[...]) + b2p)
        x3 = jnp.dot(x2, wp, preferred_element_type=jnp.float32)
        x3_ref[...] = _leaky(x3 + b_ref[...])

    return pl.pallas_call(
        body,
        out_shape=jax.ShapeDtypeStruct((NP, 10), jnp.float32),
    )(s2p, g2, dinv16, b2, fc1_W, fc1_b)


def _tc_fc2(x2d, fc2_W, fc2_b2d):
    def body(x_ref, w_ref, b_ref, out_ref):
        part = jnp.dot(x_ref[...], w_ref[...], preferred_element_type=jnp.float32)
        out_ref[...] = _leaky(part + b_ref[...])

    return pl.pallas_call(
        body,
        out_shape=jax.ShapeDtypeStruct((1, 100), jnp.float32),
    )(x2d, fc2_W, fc2_b2d)


def kernel(pos, edge_index, W1, b1, W2, b2, fc1_W, fc1_b, fc2_W, fc2_b):
    pad = N + (jnp.arange(EP - E, dtype=jnp.int32) % 8)
    srcp = jnp.concatenate([edge_index[0], pad]).reshape(NW, NCH, CH)
    dstp = jnp.concatenate([edge_index[1], pad]).reshape(NW, NCH, CH)
    posp = jnp.pad(pos, ((0, NP - N), (0, 0)))

    h1 = _tc_h1(posp, W1)
    s1p, g1, dinv16 = _sc_layer1(h1, srcp, dstp)
    g2 = _tc_mid(s1p, g1, dinv16, b1, W2)
    s2p = _sc_scatter(g2, srcp, dstp, D2)
    x3 = _tc_post(s2p, g2, dinv16, b2, fc1_W, fc1_b)
    x2d = x3[:N].reshape(1, N * 10)
    out = _tc_fc2(x2d, fc2_W, fc2_b.reshape(1, 100))
    return out.reshape(100)
